# Initial kernel scaffold; baseline (speedup 1.0000x reference)
import jax
import jax.numpy as jnp
from jax.experimental import pallas as pl


def kernel(s_parent, mu_parent, Sigma_parent, mask_parent, xi_noise, emb, ln1_g, ln1_b, W1, b1, w2, b2, ln2_g, ln2_b, W3, b3, w4, b4, Wg, bg):
    raise NotImplementedError("write your pallas kernel here")


# speedup vs baseline: 17.9305x; 17.9305x over previous
import functools

import jax
import jax.numpy as jnp
from jax.experimental import pallas as pl

_PHI = 1.6
_LAM = 1e-4
_EPS = 1e-5


def _gate_kernel(s_ref, emb_ref, g_ref, bta_ref, W1_ref, b1_ref, w2_ref,
                 b2_ref, mask_ref, mu_ref, sa_ref, sb_ref,
                 sc0_ref, gout_ref, muc_ref, sigc_ref, loss_ref,
                 *, rows, m):
    f32 = jnp.float32
    s = s_ref[...]
    gam = g_ref[...]
    emb_v = emb_ref[...]
    W1 = W1_ref[...]
    C = s.shape[1]

    ms = jnp.mean(s, axis=1, keepdims=True)
    ss = jnp.mean(s * s, axis=1, keepdims=True)
    me = jnp.mean(emb_v, axis=1, keepdims=True)
    ee = jnp.mean(emb_v * emb_v, axis=1, keepdims=True)
    cross = jax.lax.dot_general(s, emb_v, (((1,), (1,)), ((), ())),
                                preferred_element_type=f32) * (1.0 / C)
    mu_km = ms + me.T
    var = ss + 2.0 * cross + ee.T - mu_km * mu_km
    inv = jax.lax.rsqrt(var + _EPS)

    dot = functools.partial(jax.lax.dot_general,
                            dimension_numbers=(((1,), (0,)), ((), ())),
                            preferred_element_type=f32)
    P = dot(s * gam, W1)
    Q = dot(emb_v * gam, W1)
    u = dot(gam, W1)
    v = dot(bta_ref[...], W1) + b1_ref[...]

    z = (inv[:, :, None] * (P[:, None, :] + Q[None, :, :]
                            - mu_km[:, :, None] * u[None, :, :])
         + v[None, :, :])
    h = z * jax.nn.sigmoid(z)
    logit = jax.lax.dot_general(h, w2_ref[...], (((2,), (0,)), ((), ())),
                                preferred_element_type=f32)
    gq = jax.nn.sigmoid(logit[:, :, 0] + b2_ref[...]) * mask_ref[...]

    gout_ref[...] = gq
    sc0_ref[...] = gq[:, :, None] * s[:, None, :]
    muc_ref[...] = jnp.broadcast_to(mu_ref[...][:, None, :], (rows, m, 3))
    ssym = (sa_ref[...] + sb_ref[...]) * (0.5 * _PHI ** -2)
    lane = jax.lax.broadcasted_iota(jnp.int32, (rows, 9), 1)
    ssym = ssym + jnp.where(lane % 4 == 0, f32(_LAM), f32(0.0))
    sigc_ref[...] = jnp.broadcast_to(ssym[:, None, :], (rows, m, 9))
    loss_ref[...] = jnp.sum(gq, keepdims=True).reshape(1, 1) * 0.0


def kernel(s_parent, mu_parent, Sigma_parent, mask_parent, xi_noise, emb,
           ln1_g, ln1_b, W1, b1, w2, b2, ln2_g, ln2_b, W3, b3, w4, b4,
           Wg, bg):
    f32 = jnp.float32
    B, Kp, C = s_parent.shape
    M = emb.shape[0]
    N = B * Kp

    s2d = s_parent.reshape(N, C)
    mask2d = mask_parent.reshape(N, 1).astype(f32)
    mu2d = mu_parent.reshape(N, 3)
    sig9 = Sigma_parent.reshape(N, 9)
    sigT9 = jnp.swapaxes(Sigma_parent, -1, -2).reshape(N, 9)
    gam = ln1_g.reshape(1, C)
    bta = ln1_b.reshape(1, C)
    b1r = b1.reshape(1, C)
    w2r = w2.reshape(C, 1)
    b2r = b2.reshape(1, 1)

    GRID = 8
    R = N // GRID

    full = lambda shape: pl.BlockSpec(shape, lambda i: (0,) * len(shape))
    rowblk = lambda *trail: pl.BlockSpec((R,) + trail,
                                         lambda i: (i,) + (0,) * len(trail))

    out_shapes = (
        jax.ShapeDtypeStruct((N, M, C), f32),
        jax.ShapeDtypeStruct((N, M), f32),
        jax.ShapeDtypeStruct((N, M, 3), f32),
        jax.ShapeDtypeStruct((N, M, 9), f32),
        jax.ShapeDtypeStruct((1, 1), f32),
    )

    sc0, gout, muc, sigc, loss = pl.pallas_call(
        functools.partial(_gate_kernel, rows=R, m=M),
        grid=(GRID,),
        in_specs=[
            rowblk(C),
            full((M, C)),
            full((1, C)),
            full((1, C)),
            full((C, C)),
            full((1, C)),
            full((C, 1)),
            full((1, 1)),
            rowblk(1),
            rowblk(3),
            rowblk(9),
            rowblk(9),
        ],
        out_specs=[
            rowblk(M, C),
            rowblk(M),
            rowblk(M, 3),
            rowblk(M, 9),
            full((1, 1)),
        ],
        out_shape=out_shapes,
    )(s2d, emb, gam, bta, W1, b1r, w2r, b2r, mask2d, mu2d, sig9, sigT9)

    Kc = Kp * M
    s_child0 = sc0.reshape(B, Kc, C)
    g = gout.reshape(B, Kc)
    mu_child = muc.reshape(B, Kc, 3)
    Sigma_child = sigc.reshape(B, Kc, 3, 3)
    loss_count = loss.reshape(())
    return (s_child0, mu_child, Sigma_child, g, loss_count)
